# baseline (speedup 1.0000x reference)
import jax
import jax.numpy as jnp
from jax import lax
from jax.experimental import pallas as pl
from jax.experimental.pallas import tpu as pltpu
from jax.experimental.pallas import tpu_sc as plsc

N = 10000
E = 320000
D = 128

NC = 2
NS = 16
NW = NC * NS
CHUNK = 128
GRP = 8
CH_PER_W = GRP * (-(-E // (NW * CHUNK * GRP)))
E_PAD = NW * CH_PER_W * CHUNK
HALVES = 2
CPH = CH_PER_W // HALVES
ROWS_PAD = 10240
RPT = ROWS_PAD // NS


def _make_sc_agg(with_counts: bool):
    mesh = plsc.VectorSubcoreMesh(core_axis_name="c", subcore_axis_name="s",
                                  num_cores=NC, num_subcores=NS)
    out_type = [jax.ShapeDtypeStruct((NC, ROWS_PAD, D), jnp.float32)]
    scratch = [
        pltpu.VMEM((CPH, CHUNK), jnp.int32),
        pltpu.VMEM((CPH, CHUNK), jnp.int32),
        pltpu.VMEM((CHUNK, D), jnp.float32),
        pltpu.VMEM_SHARED((ROWS_PAD, D), jnp.float32),
        pltpu.SemaphoreType.DMA,
    ]
    if with_counts:
        out_type.append(jax.ShapeDtypeStruct((NC, ROWS_PAD), jnp.float32))
        scratch += [
            pltpu.VMEM((CHUNK,), jnp.float32),
            pltpu.VMEM_SHARED((ROWS_PAD,), jnp.float32),
        ]

    def body(feat_hbm, src_hbm, dst_hbm, zf_hbm, zc_hbm, *rest):
        if with_counts:
            (out_hbm, cnt_hbm, src_v, dst_v, rows_v, acc_sh, sem,
             ones_v, cnt_sh) = rest
        else:
            out_hbm, src_v, dst_v, rows_v, acc_sh, sem = rest
        cid = lax.axis_index("c")
        sid = lax.axis_index("s")
        wid = cid * NS + sid

        pltpu.sync_copy(zf_hbm.at[pl.ds(sid * RPT, RPT)],
                        acc_sh.at[pl.ds(sid * RPT, RPT)])
        if with_counts:
            pltpu.sync_copy(zc_hbm.at[pl.ds(sid * RPT, RPT)],
                            cnt_sh.at[pl.ds(sid * RPT, RPT)])
            for k in range(CHUNK // 16):
                ones_v[pl.ds(k * 16, 16)] = jnp.ones((16,), jnp.float32)
        plsc.subcore_barrier()

        def group(g, carry):
            for k in range(GRP):
                j = g * GRP + k
                pltpu.async_copy(feat_hbm.at[src_v.at[j]], rows_v, sem).wait()
                pltpu.sync_copy(rows_v, acc_sh.at[dst_v.at[j]], add=True)
                if with_counts:
                    pltpu.sync_copy(ones_v, cnt_sh.at[dst_v.at[j]], add=True)
            return carry

        for half in range(HALVES):
            pltpu.sync_copy(src_hbm.at[wid, pl.ds(half * CPH, CPH)], src_v)
            pltpu.sync_copy(dst_hbm.at[wid, pl.ds(half * CPH, CPH)], dst_v)
            lax.fori_loop(0, CPH // GRP, group, None)
        plsc.subcore_barrier()

        pltpu.sync_copy(acc_sh.at[pl.ds(sid * RPT, RPT)],
                        out_hbm.at[cid, pl.ds(sid * RPT, RPT)])
        if with_counts:
            pltpu.sync_copy(cnt_sh.at[pl.ds(sid * RPT, RPT)],
                            cnt_hbm.at[cid, pl.ds(sid * RPT, RPT)])

    return pl.kernel(body, out_type=out_type, mesh=mesh, scratch_types=scratch)


_sc_agg_counts = _make_sc_agg(True)
_sc_agg = _make_sc_agg(False)


R = 400
_f32 = jnp.float32


def _tc1_body(x_ref, p0_ref, p1_ref, c0_ref, c1_ref,
              wl1t_ref, bl1_ref, wr1t_ref, wlint_ref, blin_ref, out_ref):
    cnt = jnp.maximum(c0_ref[...] + c1_ref[...], 1.0)
    mean = (p0_ref[0] + p1_ref[0]) / cnt
    t = (jnp.dot(mean, wl1t_ref[...], preferred_element_type=_f32)
         + bl1_ref[...]
         + jnp.dot(x_ref[...], wr1t_ref[...], preferred_element_type=_f32))
    nrm = jnp.sqrt(jnp.sum(t * t, axis=1, keepdims=True))
    h1 = t / jnp.maximum(nrm, 1e-12)
    h = jnp.dot(h1, wlint_ref[...], preferred_element_type=_f32) + blin_ref[...]
    out_ref[...] = jnp.maximum(h, 0.0)


def _tc2_body(h_ref, p0_ref, p1_ref, c0_ref, c1_ref,
              wl2t_ref, bl2_ref, wr2t_ref, wlin2t_ref, blin2_ref,
              y_ref, p_ref):
    cnt = jnp.maximum(c0_ref[...] + c1_ref[...], 1.0)
    mean = (p0_ref[0] + p1_ref[0]) / cnt
    t = (jnp.dot(mean, wl2t_ref[...], preferred_element_type=_f32)
         + bl2_ref[...]
         + jnp.dot(h_ref[...], wr2t_ref[...], preferred_element_type=_f32))
    nrm = jnp.sqrt(jnp.sum(t * t, axis=1, keepdims=True))
    y = t / jnp.maximum(nrm, 1e-12)
    y_ref[...] = y
    logits = jnp.dot(y, wlin2t_ref[...], preferred_element_type=_f32) + blin2_ref[...]
    m = jnp.max(logits, axis=1, keepdims=True)
    e = jnp.exp(logits - m)
    p_ref[...] = e / jnp.sum(e, axis=1, keepdims=True)


_row_spec = pl.BlockSpec((R, D), lambda i: (i, 0))
_agg0_spec = pl.BlockSpec((1, R, D), lambda i: (0, i, 0))
_agg1_spec = pl.BlockSpec((1, R, D), lambda i: (1, i, 0))
_cnt_spec = pl.BlockSpec((R, 1), lambda i: (i, 0))
_w_spec = pl.BlockSpec((D, D), lambda i: (0, 0))
_b_spec = pl.BlockSpec((1, D), lambda i: (0, 0))

_tc1 = pl.pallas_call(
    _tc1_body,
    grid=(N // R,),
    in_specs=[_row_spec, _agg0_spec, _agg1_spec, _cnt_spec, _cnt_spec,
              _w_spec, _b_spec, _w_spec, _w_spec, _b_spec],
    out_specs=_row_spec,
    out_shape=jax.ShapeDtypeStruct((N, D), _f32),
)

_tc2 = pl.pallas_call(
    _tc2_body,
    grid=(N // R,),
    in_specs=[_row_spec, _agg0_spec, _agg1_spec, _cnt_spec, _cnt_spec,
              _w_spec, _b_spec, _w_spec, _w_spec, _b_spec],
    out_specs=[_row_spec, _row_spec],
    out_shape=[jax.ShapeDtypeStruct((N, D), _f32),
               jax.ShapeDtypeStruct((N, D), _f32)],
)


def kernel(x, edge_index, Wl1, bl1, Wr1, Wlin, blin, Wl2, bl2, Wr2, Wlin2, blin2):
    x = x.astype(jnp.float32)
    src = edge_index[0]
    dst = edge_index[1]
    pad = E_PAD - E
    srcb = jnp.concatenate([src, jnp.zeros((pad,), jnp.int32)]).reshape(
        NW, CH_PER_W, CHUNK)
    dstb = jnp.concatenate([dst, jnp.full((pad,), N, jnp.int32)]).reshape(
        NW, CH_PER_W, CHUNK)
    zf = jnp.zeros((ROWS_PAD, D), _f32)
    zc = jnp.zeros((ROWS_PAD,), _f32)

    agg1, cnt = _sc_agg_counts(x, srcb, dstb, zf, zc)
    c0 = cnt[0, :N, None]
    c1 = cnt[1, :N, None]
    h = _tc1(x, agg1, agg1, c0, c1,
             Wl1.T, bl1[None, :], Wr1.T, Wlin.T, blin[None, :])

    (agg2,) = _sc_agg(h, srcb, dstb, zf, zc)
    wlin2t = jnp.zeros((D, D), _f32).at[:, :2].set(Wlin2.T)
    blin2p = jnp.full((1, D), -1e30, _f32).at[0, :2].set(blin2)
    y, p_pad = _tc2(h, agg2, agg2, c0, c1,
                    Wl2.T, bl2[None, :], Wr2.T, wlin2t, blin2p)
    return (p_pad[:, :2], y)


# speedup vs baseline: 1.0106x; 1.0106x over previous
import jax
import jax.numpy as jnp
from jax import lax
from jax.experimental import pallas as pl
from jax.experimental.pallas import tpu as pltpu
from jax.experimental.pallas import tpu_sc as plsc

N = 10000
E = 320000
D = 128

NC = 2
NS = 16
NW = NC * NS
CHUNK = 128
GRP = 8
CH_PER_W = GRP * (-(-E // (NW * CHUNK * GRP)))
E_PAD = NW * CH_PER_W * CHUNK
HALVES = 2
CPH = CH_PER_W // HALVES
ROWS_PAD = 10240
RPT = ROWS_PAD // NS


def _make_sc_agg(with_counts: bool):
    mesh = plsc.VectorSubcoreMesh(core_axis_name="c", subcore_axis_name="s",
                                  num_cores=NC, num_subcores=NS)
    out_type = [jax.ShapeDtypeStruct((NC, ROWS_PAD, D), jnp.float32)]
    scratch = [
        pltpu.VMEM((CH_PER_W, CHUNK), jnp.int32),
        pltpu.VMEM((CH_PER_W, CHUNK), jnp.int32),
        pltpu.VMEM((CHUNK, D), jnp.float32),
        pltpu.VMEM_SHARED((ROWS_PAD, D), jnp.float32),
        pltpu.SemaphoreType.DMA,
    ]
    if with_counts:
        out_type.append(jax.ShapeDtypeStruct((NC, ROWS_PAD), jnp.float32))
        scratch += [
            pltpu.VMEM((CHUNK,), jnp.float32),
            pltpu.VMEM_SHARED((ROWS_PAD,), jnp.float32),
        ]

    def body(feat_hbm, src_hbm, dst_hbm, zf_hbm, zc_hbm, *rest):
        if with_counts:
            (out_hbm, cnt_hbm, src_v, dst_v, rows_v, acc_sh, sem,
             ones_v, cnt_sh) = rest
        else:
            out_hbm, src_v, dst_v, rows_v, acc_sh, sem = rest
        cid = lax.axis_index("c")
        sid = lax.axis_index("s")
        wid = cid * NS + sid

        pltpu.sync_copy(zf_hbm.at[pl.ds(sid * RPT, RPT)],
                        acc_sh.at[pl.ds(sid * RPT, RPT)])
        if with_counts:
            pltpu.sync_copy(zc_hbm.at[pl.ds(sid * RPT, RPT)],
                            cnt_sh.at[pl.ds(sid * RPT, RPT)])
            for k in range(CHUNK // 16):
                ones_v[pl.ds(k * 16, 16)] = jnp.ones((16,), jnp.float32)
        plsc.subcore_barrier()

        pltpu.sync_copy(src_hbm.at[wid], src_v)
        pltpu.sync_copy(dst_hbm.at[wid], dst_v)

        def step(j, carry):
            pltpu.async_copy(feat_hbm.at[src_v.at[j]], rows_v, sem).wait()
            pltpu.sync_copy(rows_v, acc_sh.at[dst_v.at[j]], add=True)
            if with_counts:
                pltpu.sync_copy(ones_v, cnt_sh.at[dst_v.at[j]], add=True)
            return carry

        lax.fori_loop(0, CH_PER_W, step, None)
        plsc.subcore_barrier()

        pltpu.sync_copy(acc_sh.at[pl.ds(sid * RPT, RPT)],
                        out_hbm.at[cid, pl.ds(sid * RPT, RPT)])
        if with_counts:
            pltpu.sync_copy(cnt_sh.at[pl.ds(sid * RPT, RPT)],
                            cnt_hbm.at[cid, pl.ds(sid * RPT, RPT)])

    return pl.kernel(body, out_type=out_type, mesh=mesh, scratch_types=scratch)


_sc_agg_counts = _make_sc_agg(True)
_sc_agg = _make_sc_agg(False)


R = 400
_f32 = jnp.float32


def _tc1_body(x_ref, p0_ref, p1_ref, c0_ref, c1_ref,
              wl1t_ref, bl1_ref, wr1t_ref, wlint_ref, blin_ref, out_ref):
    cnt = jnp.maximum(c0_ref[...] + c1_ref[...], 1.0)
    mean = (p0_ref[0] + p1_ref[0]) / cnt
    t = (jnp.dot(mean, wl1t_ref[...], preferred_element_type=_f32)
         + bl1_ref[...]
         + jnp.dot(x_ref[...], wr1t_ref[...], preferred_element_type=_f32))
    nrm = jnp.sqrt(jnp.sum(t * t, axis=1, keepdims=True))
    h1 = t / jnp.maximum(nrm, 1e-12)
    h = jnp.dot(h1, wlint_ref[...], preferred_element_type=_f32) + blin_ref[...]
    out_ref[...] = jnp.maximum(h, 0.0)


def _tc2_body(h_ref, p0_ref, p1_ref, c0_ref, c1_ref,
              wl2t_ref, bl2_ref, wr2t_ref, wlin2t_ref, blin2_ref,
              y_ref, p_ref):
    cnt = jnp.maximum(c0_ref[...] + c1_ref[...], 1.0)
    mean = (p0_ref[0] + p1_ref[0]) / cnt
    t = (jnp.dot(mean, wl2t_ref[...], preferred_element_type=_f32)
         + bl2_ref[...]
         + jnp.dot(h_ref[...], wr2t_ref[...], preferred_element_type=_f32))
    nrm = jnp.sqrt(jnp.sum(t * t, axis=1, keepdims=True))
    y = t / jnp.maximum(nrm, 1e-12)
    y_ref[...] = y
    logits = jnp.dot(y, wlin2t_ref[...], preferred_element_type=_f32) + blin2_ref[...]
    m = jnp.max(logits, axis=1, keepdims=True)
    e = jnp.exp(logits - m)
    p_ref[...] = e / jnp.sum(e, axis=1, keepdims=True)


_row_spec = pl.BlockSpec((R, D), lambda i: (i, 0))
_agg0_spec = pl.BlockSpec((1, R, D), lambda i: (0, i, 0))
_agg1_spec = pl.BlockSpec((1, R, D), lambda i: (1, i, 0))
_cnt_spec = pl.BlockSpec((R, 1), lambda i: (i, 0))
_w_spec = pl.BlockSpec((D, D), lambda i: (0, 0))
_b_spec = pl.BlockSpec((1, D), lambda i: (0, 0))

_tc1 = pl.pallas_call(
    _tc1_body,
    grid=(N // R,),
    in_specs=[_row_spec, _agg0_spec, _agg1_spec, _cnt_spec, _cnt_spec,
              _w_spec, _b_spec, _w_spec, _w_spec, _b_spec],
    out_specs=_row_spec,
    out_shape=jax.ShapeDtypeStruct((N, D), _f32),
)

_tc2 = pl.pallas_call(
    _tc2_body,
    grid=(N // R,),
    in_specs=[_row_spec, _agg0_spec, _agg1_spec, _cnt_spec, _cnt_spec,
              _w_spec, _b_spec, _w_spec, _w_spec, _b_spec],
    out_specs=[_row_spec, _row_spec],
    out_shape=[jax.ShapeDtypeStruct((N, D), _f32),
               jax.ShapeDtypeStruct((N, D), _f32)],
)


def kernel(x, edge_index, Wl1, bl1, Wr1, Wlin, blin, Wl2, bl2, Wr2, Wlin2, blin2):
    x = x.astype(jnp.float32)
    src = edge_index[0]
    dst = edge_index[1]
    pad = E_PAD - E
    srcb = jnp.concatenate([src, jnp.zeros((pad,), jnp.int32)]).reshape(
        NW, CH_PER_W, CHUNK)
    dstb = jnp.concatenate([dst, jnp.full((pad,), N, jnp.int32)]).reshape(
        NW, CH_PER_W, CHUNK)
    zf = jnp.zeros((ROWS_PAD, D), _f32)
    zc = jnp.zeros((ROWS_PAD,), _f32)

    agg1, cnt = _sc_agg_counts(x, srcb, dstb, zf, zc)
    c0 = cnt[0, :N, None]
    c1 = cnt[1, :N, None]
    h = _tc1(x, agg1, agg1, c0, c1,
             Wl1.T, bl1[None, :], Wr1.T, Wlin.T, blin[None, :])

    (agg2,) = _sc_agg(h, srcb, dstb, zf, zc)
    wlin2t = jnp.zeros((D, D), _f32).at[:, :2].set(Wlin2.T)
    blin2p = jnp.full((1, D), -1e30, _f32).at[0, :2].set(blin2)
    y, p_pad = _tc2(h, agg2, agg2, c0, c1,
                    Wl2.T, bl2[None, :], Wr2.T, wlin2t, blin2p)
    return (p_pad[:, :2], y)


# speedup vs baseline: 1.4761x; 1.4606x over previous
import functools

import jax
import jax.numpy as jnp
from jax import lax
from jax.experimental import pallas as pl
from jax.experimental.pallas import tpu as pltpu
from jax.experimental.pallas import tpu_sc as plsc

N = 10000
E = 320000
D = 128

NC = 2
NS = 16
NW = NC * NS
CHUNK = 128
CH_PER_W = -(-E // (NW * CHUNK))
E_PAD = NW * CH_PER_W * CHUNK
ROWS_PAD = 10240
RPT = ROWS_PAD // NS


def _make_sc_agg(with_counts: bool):
    mesh = plsc.VectorSubcoreMesh(core_axis_name="c", subcore_axis_name="s",
                                  num_cores=NC, num_subcores=NS)
    out_type = [jax.ShapeDtypeStruct((NC, ROWS_PAD, D), jnp.float32)]
    scratch = [
        pltpu.VMEM((CH_PER_W, CHUNK), jnp.int32),
        pltpu.VMEM((CH_PER_W, CHUNK), jnp.int32),
        pltpu.VMEM((CHUNK, D), jnp.float32),
        pltpu.VMEM_SHARED((ROWS_PAD, D), jnp.float32),
        pltpu.SemaphoreType.DMA,
    ]
    if with_counts:
        out_type.append(jax.ShapeDtypeStruct((NC, ROWS_PAD), jnp.float32))
        scratch += [
            pltpu.VMEM((CHUNK,), jnp.float32),
            pltpu.VMEM_SHARED((ROWS_PAD,), jnp.float32),
        ]

    def body(feat_hbm, src_hbm, dst_hbm, zf_hbm, zc_hbm, *rest):
        if with_counts:
            out_hbm, cnt_hbm, src_v, dst_v, rows_v, acc_sh, sem, ones_v, cnt_sh = rest
        else:
            out_hbm, src_v, dst_v, rows_v, acc_sh, sem = rest
        cid = lax.axis_index("c")
        sid = lax.axis_index("s")
        wid = cid * NS + sid

        pltpu.sync_copy(zf_hbm.at[pl.ds(sid * RPT, RPT)],
                        acc_sh.at[pl.ds(sid * RPT, RPT)])
        pltpu.sync_copy(src_hbm.at[wid], src_v)
        pltpu.sync_copy(dst_hbm.at[wid], dst_v)
        if with_counts:
            pltpu.sync_copy(zc_hbm.at[pl.ds(sid * RPT, RPT)],
                            cnt_sh.at[pl.ds(sid * RPT, RPT)])
            for k in range(CHUNK // 16):
                ones_v[pl.ds(k * 16, 16)] = jnp.ones((16,), jnp.float32)
        plsc.subcore_barrier()

        def step(j, carry):
            pltpu.async_copy(feat_hbm.at[src_v.at[j]], rows_v, sem).wait()
            pltpu.sync_copy(rows_v, acc_sh.at[dst_v.at[j]], add=True)
            if with_counts:
                pltpu.sync_copy(ones_v, cnt_sh.at[dst_v.at[j]], add=True)
            return carry

        lax.fori_loop(0, CH_PER_W, step, None)
        plsc.subcore_barrier()

        pltpu.sync_copy(acc_sh.at[pl.ds(sid * RPT, RPT)],
                        out_hbm.at[cid, pl.ds(sid * RPT, RPT)])
        if with_counts:
            pltpu.sync_copy(cnt_sh.at[pl.ds(sid * RPT, RPT)],
                            cnt_hbm.at[cid, pl.ds(sid * RPT, RPT)])

    return pl.kernel(body, out_type=out_type, mesh=mesh, scratch_types=scratch)


_sc_agg_counts = _make_sc_agg(True)
_sc_agg = _make_sc_agg(False)


R = 400
_f32 = jnp.float32


def _tc1_body(x_ref, p0_ref, p1_ref, c0_ref, c1_ref,
              wl1t_ref, bl1_ref, wr1t_ref, wlint_ref, blin_ref, out_ref):
    cnt = jnp.maximum(c0_ref[...] + c1_ref[...], 1.0)
    mean = (p0_ref[...] + p1_ref[...]) / cnt
    t = (jnp.dot(mean, wl1t_ref[...], preferred_element_type=_f32)
         + bl1_ref[...]
         + jnp.dot(x_ref[...], wr1t_ref[...], preferred_element_type=_f32))
    nrm = jnp.sqrt(jnp.sum(t * t, axis=1, keepdims=True))
    h1 = t / jnp.maximum(nrm, 1e-12)
    h = jnp.dot(h1, wlint_ref[...], preferred_element_type=_f32) + blin_ref[...]
    out_ref[...] = jnp.maximum(h, 0.0)


def _tc2_body(h_ref, p0_ref, p1_ref, c0_ref, c1_ref,
              wl2t_ref, bl2_ref, wr2t_ref, wlin2t_ref, blin2_ref,
              y_ref, p_ref):
    cnt = jnp.maximum(c0_ref[...] + c1_ref[...], 1.0)
    mean = (p0_ref[...] + p1_ref[...]) / cnt
    t = (jnp.dot(mean, wl2t_ref[...], preferred_element_type=_f32)
         + bl2_ref[...]
         + jnp.dot(h_ref[...], wr2t_ref[...], preferred_element_type=_f32))
    nrm = jnp.sqrt(jnp.sum(t * t, axis=1, keepdims=True))
    y = t / jnp.maximum(nrm, 1e-12)
    y_ref[...] = y
    logits = jnp.dot(y, wlin2t_ref[...], preferred_element_type=_f32) + blin2_ref[...]
    m = jnp.max(logits, axis=1, keepdims=True)
    e = jnp.exp(logits - m)
    p_ref[...] = e / jnp.sum(e, axis=1, keepdims=True)


_row_spec = pl.BlockSpec((R, D), lambda i: (i, 0))
_col_spec = pl.BlockSpec((R, 1), lambda i: (i, 0))
_w_spec = pl.BlockSpec((D, D), lambda i: (0, 0))
_b_spec = pl.BlockSpec((1, D), lambda i: (0, 0))

_tc1 = pl.pallas_call(
    _tc1_body,
    grid=(N // R,),
    in_specs=[_row_spec, _row_spec, _row_spec, _col_spec, _col_spec,
              _w_spec, _b_spec, _w_spec, _w_spec, _b_spec],
    out_specs=_row_spec,
    out_shape=jax.ShapeDtypeStruct((N, D), _f32),
)

_tc2 = pl.pallas_call(
    _tc2_body,
    grid=(N // R,),
    in_specs=[_row_spec, _row_spec, _row_spec, _col_spec, _col_spec,
              _w_spec, _b_spec, _w_spec, _w_spec, _b_spec],
    out_specs=[_row_spec, _row_spec],
    out_shape=[jax.ShapeDtypeStruct((N, D), _f32),
               jax.ShapeDtypeStruct((N, D), _f32)],
)


def kernel(x, edge_index, Wl1, bl1, Wr1, Wlin, blin, Wl2, bl2, Wr2, Wlin2, blin2):
    x = x.astype(jnp.float32)
    src = edge_index[0]
    dst = edge_index[1]
    pad = E_PAD - E
    srcb = jnp.concatenate([src, jnp.zeros((pad,), jnp.int32)]).reshape(
        NW, CH_PER_W, CHUNK)
    dstb = jnp.concatenate([dst, jnp.full((pad,), N, jnp.int32)]).reshape(
        NW, CH_PER_W, CHUNK)
    zf = jnp.zeros((ROWS_PAD, D), _f32)
    zc = jnp.zeros((ROWS_PAD,), _f32)

    agg1, cnt = _sc_agg_counts(x, srcb, dstb, zf, zc)
    c0 = cnt[0, :N, None]
    c1 = cnt[1, :N, None]
    h = _tc1(x, agg1[0, :N], agg1[1, :N], c0, c1,
             Wl1.T, bl1[None, :], Wr1.T, Wlin.T, blin[None, :])

    (agg2,) = _sc_agg(h, srcb, dstb, zf, zc)
    wlin2t = jnp.zeros((D, D), _f32).at[:, :2].set(Wlin2.T)
    blin2p = jnp.full((1, D), -1e30, _f32).at[0, :2].set(blin2)
    y, p_pad = _tc2(h, agg2[0, :N], agg2[1, :N], c0, c1,
                    Wl2.T, bl2[None, :], Wr2.T, wlin2t, blin2p)
    return (p_pad[:, :2], y)


# speedup vs baseline: 1.4798x; 1.0025x over previous
import functools

import jax
import jax.numpy as jnp
from jax import lax
from jax.experimental import pallas as pl
from jax.experimental.pallas import tpu as pltpu
from jax.experimental.pallas import tpu_sc as plsc

N = 10000
E = 320000
D = 128

NC = 2
NS = 16
NW = NC * NS
CHUNK = 128
CH_PER_W = -(-E // (NW * CHUNK))
E_PAD = NW * CH_PER_W * CHUNK
ROWS_PAD = 10240
RPT = ROWS_PAD // NS


def _make_sc_agg(with_counts: bool):
    mesh = plsc.VectorSubcoreMesh(core_axis_name="c", subcore_axis_name="s",
                                  num_cores=NC, num_subcores=NS)
    out_type = [jax.ShapeDtypeStruct((NC, ROWS_PAD, D), jnp.float32)]
    scratch = [
        pltpu.VMEM((CH_PER_W, CHUNK), jnp.int32),
        pltpu.VMEM((CH_PER_W, CHUNK), jnp.int32),
        pltpu.VMEM((CHUNK, D), jnp.float32),
        pltpu.VMEM_SHARED((ROWS_PAD, D), jnp.float32),
        pltpu.SemaphoreType.DMA,
    ]
    if with_counts:
        out_type.append(jax.ShapeDtypeStruct((NC, ROWS_PAD), jnp.float32))
        scratch += [
            pltpu.VMEM((CHUNK,), jnp.float32),
            pltpu.VMEM_SHARED((ROWS_PAD,), jnp.float32),
        ]

    def body(feat_hbm, src_hbm, dst_hbm, zf_hbm, zc_hbm, *rest):
        if with_counts:
            out_hbm, cnt_hbm, src_v, dst_v, rows_v, acc_sh, sem, ones_v, cnt_sh = rest
        else:
            out_hbm, src_v, dst_v, rows_v, acc_sh, sem = rest
        cid = lax.axis_index("c")
        sid = lax.axis_index("s")
        wid = cid * NS + sid

        pltpu.sync_copy(zf_hbm.at[pl.ds(sid * RPT, RPT)],
                        acc_sh.at[pl.ds(sid * RPT, RPT)])
        pltpu.sync_copy(src_hbm.at[wid], src_v)
        pltpu.sync_copy(dst_hbm.at[wid], dst_v)
        if with_counts:
            pltpu.sync_copy(zc_hbm.at[pl.ds(sid * RPT, RPT)],
                            cnt_sh.at[pl.ds(sid * RPT, RPT)])
            for k in range(CHUNK // 16):
                ones_v[pl.ds(k * 16, 16)] = jnp.ones((16,), jnp.float32)
        plsc.subcore_barrier()

        def step(j, carry):
            pltpu.async_copy(feat_hbm.at[src_v.at[j]], rows_v, sem).wait()
            pltpu.sync_copy(rows_v, acc_sh.at[dst_v.at[j]], add=True)
            if with_counts:
                pltpu.sync_copy(ones_v, cnt_sh.at[dst_v.at[j]], add=True)
            return carry

        lax.fori_loop(0, CH_PER_W, step, None)
        plsc.subcore_barrier()

        pltpu.sync_copy(acc_sh.at[pl.ds(sid * RPT, RPT)],
                        out_hbm.at[cid, pl.ds(sid * RPT, RPT)])
        if with_counts:
            pltpu.sync_copy(cnt_sh.at[pl.ds(sid * RPT, RPT)],
                            cnt_hbm.at[cid, pl.ds(sid * RPT, RPT)])

    return pl.kernel(body, out_type=out_type, mesh=mesh, scratch_types=scratch)


_sc_agg_counts = _make_sc_agg(True)
_sc_agg = _make_sc_agg(False)


R = 400
_f32 = jnp.float32


def _tc1_body(x_ref, p0_ref, p1_ref, c0_ref, c1_ref,
              wl1t_ref, bl1_ref, wr1t_ref, wlint_ref, blin_ref, out_ref):
    cnt = jnp.maximum(c0_ref[...] + c1_ref[...], 1.0)
    mean = (p0_ref[...] + p1_ref[...]) / cnt
    t = (jnp.dot(mean, wl1t_ref[...], preferred_element_type=_f32)
         + bl1_ref[...]
         + jnp.dot(x_ref[...], wr1t_ref[...], preferred_element_type=_f32))
    nrm = jnp.sqrt(jnp.sum(t * t, axis=1, keepdims=True))
    h1 = t / jnp.maximum(nrm, 1e-12)
    h = jnp.dot(h1, wlint_ref[...], preferred_element_type=_f32) + blin_ref[...]
    out_ref[...] = jnp.maximum(h, 0.0)


def _tc2_body(h_ref, p0_ref, p1_ref, c0_ref, c1_ref,
              wl2t_ref, bl2_ref, wr2t_ref, wlin2t_ref, blin2_ref,
              y_ref, p_ref):
    cnt = jnp.maximum(c0_ref[...] + c1_ref[...], 1.0)
    mean = (p0_ref[...] + p1_ref[...]) / cnt
    t = (jnp.dot(mean, wl2t_ref[...], preferred_element_type=_f32)
         + bl2_ref[...]
         + jnp.dot(h_ref[...], wr2t_ref[...], preferred_element_type=_f32))
    nrm = jnp.sqrt(jnp.sum(t * t, axis=1, keepdims=True))
    y = t / jnp.maximum(nrm, 1e-12)
    y_ref[...] = y
    logits = jnp.dot(y, wlin2t_ref[...], preferred_element_type=_f32) + blin2_ref[...]
    m = jnp.max(logits, axis=1, keepdims=True)
    e = jnp.exp(logits - m)
    p_ref[...] = e / jnp.sum(e, axis=1, keepdims=True)


_row_spec = pl.BlockSpec((R, D), lambda i: (i, 0))
_col_spec = pl.BlockSpec((R, 1), lambda i: (i, 0))
_w_spec = pl.BlockSpec((D, D), lambda i: (0, 0))
_b_spec = pl.BlockSpec((1, D), lambda i: (0, 0))

_tc1 = pl.pallas_call(
    _tc1_body,
    grid=(N // R,),
    in_specs=[_row_spec, _row_spec, _row_spec, _col_spec, _col_spec,
              _w_spec, _b_spec, _w_spec, _w_spec, _b_spec],
    out_specs=_row_spec,
    out_shape=jax.ShapeDtypeStruct((N, D), _f32),
)

_tc2 = pl.pallas_call(
    _tc2_body,
    grid=(N // R,),
    in_specs=[_row_spec, _row_spec, _row_spec, _col_spec, _col_spec,
              _w_spec, _b_spec, _w_spec, _w_spec, _b_spec],
    out_specs=[_row_spec, _row_spec],
    out_shape=[jax.ShapeDtypeStruct((N, D), _f32),
               jax.ShapeDtypeStruct((N, D), _f32)],
)


def kernel(x, edge_index, Wl1, bl1, Wr1, Wlin, blin, Wl2, bl2, Wr2, Wlin2, blin2):
    x = x.astype(jnp.float32)
    src = edge_index[0]
    dst = edge_index[1]
    pad = E_PAD - E
    srcb = jnp.concatenate([src, jnp.zeros((pad,), jnp.int32)]).reshape(
        NW, CH_PER_W, CHUNK)
    dpad = N + (jnp.arange(pad, dtype=jnp.int32) % (ROWS_PAD - N))
    dstb = jnp.concatenate([dst, dpad]).reshape(NW, CH_PER_W, CHUNK)
    zf = jnp.zeros((ROWS_PAD, D), _f32)
    zc = jnp.zeros((ROWS_PAD,), _f32)

    agg1, cnt = _sc_agg_counts(x, srcb, dstb, zf, zc)
    c0 = cnt[0, :N, None]
    c1 = cnt[1, :N, None]
    h = _tc1(x, agg1[0, :N], agg1[1, :N], c0, c1,
             Wl1.T, bl1[None, :], Wr1.T, Wlin.T, blin[None, :])

    (agg2,) = _sc_agg(h, srcb, dstb, zf, zc)
    wlin2t = jnp.zeros((D, D), _f32).at[:, :2].set(Wlin2.T)
    blin2p = jnp.full((1, D), -1e30, _f32).at[0, :2].set(blin2)
    y, p_pad = _tc2(h, agg2[0, :N], agg2[1, :N], c0, c1,
                    Wl2.T, bl2[None, :], Wr2.T, wlin2t, blin2p)
    return (p_pad[:, :2], y)


# speedup vs baseline: 2.5337x; 1.7122x over previous
import functools

import jax
import jax.numpy as jnp
from jax import lax
from jax.experimental import pallas as pl
from jax.experimental.pallas import tpu as pltpu
from jax.experimental.pallas import tpu_sc as plsc

N = 10000
E = 320000
D = 128

NC = 2
NS = 16
NW = NC * NS
CHUNK = 128
CH_PER_W = -(-E // (NW * CHUNK))
E_PAD = NW * CH_PER_W * CHUNK
ROWS_PAD = 10240
RPT = ROWS_PAD // NS


def _make_sc_agg(with_counts: bool):
    mesh = plsc.VectorSubcoreMesh(core_axis_name="c", subcore_axis_name="s",
                                  num_cores=NC, num_subcores=NS)
    out_type = [jax.ShapeDtypeStruct((NC, ROWS_PAD, D), jnp.float32)]
    scratch = [
        pltpu.VMEM((CH_PER_W, CHUNK), jnp.int32),
        pltpu.VMEM((CH_PER_W, CHUNK), jnp.int32),
        pltpu.VMEM((CHUNK, D), jnp.float32),
        pltpu.VMEM_SHARED((ROWS_PAD, D), jnp.float32),
        pltpu.SemaphoreType.DMA,
    ]
    if with_counts:
        out_type.append(jax.ShapeDtypeStruct((NC, ROWS_PAD), jnp.float32))
        scratch += [
            pltpu.VMEM((CHUNK,), jnp.float32),
            pltpu.VMEM_SHARED((ROWS_PAD,), jnp.float32),
        ]

    def body(feat_hbm, src_hbm, dst_hbm, zf_hbm, zc_hbm, *rest):
        if with_counts:
            out_hbm, cnt_hbm, src_v, dst_v, rows_v, acc_sh, sem, ones_v, cnt_sh = rest
        else:
            out_hbm, src_v, dst_v, rows_v, acc_sh, sem = rest
        cid = lax.axis_index("c")
        sid = lax.axis_index("s")
        wid = cid * NS + sid

        pltpu.sync_copy(zf_hbm.at[pl.ds(sid * RPT, RPT)],
                        acc_sh.at[pl.ds(sid * RPT, RPT)])
        pltpu.sync_copy(src_hbm.at[wid], src_v)
        pltpu.sync_copy(dst_hbm.at[wid], dst_v)
        if with_counts:
            pltpu.sync_copy(zc_hbm.at[pl.ds(sid * RPT, RPT)],
                            cnt_sh.at[pl.ds(sid * RPT, RPT)])
            for k in range(CHUNK // 16):
                ones_v[pl.ds(k * 16, 16)] = jnp.ones((16,), jnp.float32)
        plsc.subcore_barrier()

        def step(j, carry):
            pltpu.async_copy(feat_hbm.at[src_v.at[j]], rows_v, sem).wait()
            pltpu.sync_copy(rows_v, acc_sh.at[dst_v.at[j]], add=True)
            if with_counts:
                pltpu.sync_copy(ones_v, cnt_sh.at[dst_v.at[j]], add=True)
            return carry

        lax.fori_loop(0, CH_PER_W, step, None)
        plsc.subcore_barrier()

        pltpu.sync_copy(acc_sh.at[pl.ds(sid * RPT, RPT)],
                        out_hbm.at[cid, pl.ds(sid * RPT, RPT)])
        if with_counts:
            pltpu.sync_copy(cnt_sh.at[pl.ds(sid * RPT, RPT)],
                            cnt_hbm.at[cid, pl.ds(sid * RPT, RPT)])

    return pl.kernel(body, out_type=out_type, mesh=mesh, scratch_types=scratch)


_sc_agg_counts = _make_sc_agg(True)
_sc_agg = _make_sc_agg(False)


R = 400
_f32 = jnp.float32


def _tc1_body(x_ref, p0_ref, p1_ref, c0_ref, c1_ref,
              wl1t_ref, bl1_ref, wr1t_ref, wlint_ref, blin_ref, out_ref):
    cnt = jnp.maximum(c0_ref[...] + c1_ref[...], 1.0)
    mean = (p0_ref[...] + p1_ref[...]) / cnt
    t = (jnp.dot(mean, wl1t_ref[...], preferred_element_type=_f32)
         + bl1_ref[...]
         + jnp.dot(x_ref[...], wr1t_ref[...], preferred_element_type=_f32))
    nrm = jnp.sqrt(jnp.sum(t * t, axis=1, keepdims=True))
    h1 = t / jnp.maximum(nrm, 1e-12)
    h = jnp.dot(h1, wlint_ref[...], preferred_element_type=_f32) + blin_ref[...]
    out_ref[...] = jnp.maximum(h, 0.0)


def _tc2_body(h_ref, p0_ref, p1_ref, c0_ref, c1_ref,
              wl2t_ref, bl2_ref, wr2t_ref, wlin2t_ref, blin2_ref,
              y_ref, p_ref):
    cnt = jnp.maximum(c0_ref[...] + c1_ref[...], 1.0)
    mean = (p0_ref[...] + p1_ref[...]) / cnt
    t = (jnp.dot(mean, wl2t_ref[...], preferred_element_type=_f32)
         + bl2_ref[...]
         + jnp.dot(h_ref[...], wr2t_ref[...], preferred_element_type=_f32))
    nrm = jnp.sqrt(jnp.sum(t * t, axis=1, keepdims=True))
    y = t / jnp.maximum(nrm, 1e-12)
    y_ref[...] = y
    logits = jnp.dot(y, wlin2t_ref[...], preferred_element_type=_f32) + blin2_ref[...]
    m = jnp.max(logits, axis=1, keepdims=True)
    e = jnp.exp(logits - m)
    p_ref[...] = e / jnp.sum(e, axis=1, keepdims=True)


_row_spec = pl.BlockSpec((R, D), lambda i: (i, 0))
_col_spec = pl.BlockSpec((R, 1), lambda i: (i, 0))
_w_spec = pl.BlockSpec((D, D), lambda i: (0, 0))
_b_spec = pl.BlockSpec((1, D), lambda i: (0, 0))

_tc1 = pl.pallas_call(
    _tc1_body,
    grid=(N // R,),
    in_specs=[_row_spec, _row_spec, _row_spec, _col_spec, _col_spec,
              _w_spec, _b_spec, _w_spec, _w_spec, _b_spec],
    out_specs=_row_spec,
    out_shape=jax.ShapeDtypeStruct((N, D), _f32),
)

_tc2 = pl.pallas_call(
    _tc2_body,
    grid=(N // R,),
    in_specs=[_row_spec, _row_spec, _row_spec, _col_spec, _col_spec,
              _w_spec, _b_spec, _w_spec, _w_spec, _b_spec],
    out_specs=[_row_spec, _row_spec],
    out_shape=[jax.ShapeDtypeStruct((N, D), _f32),
               jax.ShapeDtypeStruct((N, D), _f32)],
)


def kernel(x, edge_index, Wl1, bl1, Wr1, Wlin, blin, Wl2, bl2, Wr2, Wlin2, blin2):
    x = x.astype(jnp.float32)
    src = edge_index[0]
    dst = edge_index[1]
    pad = E_PAD - E
    spad = jnp.arange(pad, dtype=jnp.int32) * 37 % N
    srcb = jnp.concatenate([src, spad]).reshape(NW, CH_PER_W, CHUNK)
    dpad = N + (jnp.arange(pad, dtype=jnp.int32) % (ROWS_PAD - N))
    dstb = jnp.concatenate([dst, dpad]).reshape(NW, CH_PER_W, CHUNK)
    zf = jnp.zeros((ROWS_PAD, D), _f32)
    zc = jnp.zeros((ROWS_PAD,), _f32)

    agg1, cnt = _sc_agg_counts(x, srcb, dstb, zf, zc)
    c0 = cnt[0, :N, None]
    c1 = cnt[1, :N, None]
    h = _tc1(x, agg1[0, :N], agg1[1, :N], c0, c1,
             Wl1.T, bl1[None, :], Wr1.T, Wlin.T, blin[None, :])

    (agg2,) = _sc_agg(h, srcb, dstb, zf, zc)
    wlin2t = jnp.zeros((D, D), _f32).at[:, :2].set(Wlin2.T)
    blin2p = jnp.full((1, D), -1e30, _f32).at[0, :2].set(blin2)
    y, p_pad = _tc2(h, agg2[0, :N], agg2[1, :N], c0, c1,
                    Wl2.T, bl2[None, :], Wr2.T, wlin2t, blin2p)
    return (p_pad[:, :2], y)


# speedup vs baseline: 3.5028x; 1.3825x over previous
import functools

import jax
import jax.numpy as jnp
from jax import lax
from jax.experimental import pallas as pl
from jax.experimental.pallas import tpu as pltpu
from jax.experimental.pallas import tpu_sc as plsc

N = 10000
E = 320000
D = 128

NC = 2
NS = 16
NW = NC * NS
CHUNK = 128
CH_PER_W = 2 * (-(-E // (NW * CHUNK * 2)))
E_PAD = NW * CH_PER_W * CHUNK
HALVES = 2
CPH = CH_PER_W // HALVES
ROWS_PAD = 10240
RPT = ROWS_PAD // NS


def _make_sc_agg(with_counts: bool):
    mesh = plsc.VectorSubcoreMesh(core_axis_name="c", subcore_axis_name="s",
                                  num_cores=NC, num_subcores=NS)
    out_type = [jax.ShapeDtypeStruct((NC, ROWS_PAD, D), jnp.float32)]
    scratch = [
        pltpu.VMEM((CPH, CHUNK), jnp.int32),
        pltpu.VMEM((CPH, CHUNK), jnp.int32),
        pltpu.VMEM((CHUNK, D), jnp.float32),
        pltpu.VMEM((CHUNK, D), jnp.float32),
        pltpu.VMEM_SHARED((ROWS_PAD, D), jnp.float32),
        pltpu.SemaphoreType.DMA,
        pltpu.SemaphoreType.DMA,
    ]
    if with_counts:
        out_type.append(jax.ShapeDtypeStruct((NC, ROWS_PAD), jnp.float32))
        scratch += [
            pltpu.VMEM((CHUNK,), jnp.float32),
            pltpu.VMEM_SHARED((ROWS_PAD,), jnp.float32),
        ]

    def body(feat_hbm, src_hbm, dst_hbm, zf_hbm, zc_hbm, *rest):
        if with_counts:
            (out_hbm, cnt_hbm, src_v, dst_v, rows0, rows1, acc_sh, sem0, sem1,
             ones_v, cnt_sh) = rest
        else:
            out_hbm, src_v, dst_v, rows0, rows1, acc_sh, sem0, sem1 = rest
        cid = lax.axis_index("c")
        sid = lax.axis_index("s")
        wid = cid * NS + sid

        pltpu.sync_copy(zf_hbm.at[pl.ds(sid * RPT, RPT)],
                        acc_sh.at[pl.ds(sid * RPT, RPT)])
        if with_counts:
            pltpu.sync_copy(zc_hbm.at[pl.ds(sid * RPT, RPT)],
                            cnt_sh.at[pl.ds(sid * RPT, RPT)])
            for k in range(CHUNK // 16):
                ones_v[pl.ds(k * 16, 16)] = jnp.ones((16,), jnp.float32)
        plsc.subcore_barrier()

        def step(i, carry):
            j = 2 * i
            pltpu.async_copy(feat_hbm.at[src_v.at[j + 1]], rows1, sem1)
            pltpu.make_async_copy(feat_hbm.at[src_v.at[j]], rows0, sem0).wait()
            pltpu.sync_copy(rows0, acc_sh.at[dst_v.at[j]], add=True)
            if with_counts:
                pltpu.sync_copy(ones_v, cnt_sh.at[dst_v.at[j]], add=True)
            nxt = jnp.minimum(j + 2, CPH - 2)
            pltpu.async_copy(feat_hbm.at[src_v.at[nxt]], rows0, sem0)
            pltpu.make_async_copy(feat_hbm.at[src_v.at[j + 1]], rows1, sem1).wait()
            pltpu.sync_copy(rows1, acc_sh.at[dst_v.at[j + 1]], add=True)
            if with_counts:
                pltpu.sync_copy(ones_v, cnt_sh.at[dst_v.at[j + 1]], add=True)
            return carry

        for half in range(HALVES):
            pltpu.sync_copy(src_hbm.at[wid, pl.ds(half * CPH, CPH)], src_v)
            pltpu.sync_copy(dst_hbm.at[wid, pl.ds(half * CPH, CPH)], dst_v)
            pltpu.async_copy(feat_hbm.at[src_v.at[0]], rows0, sem0)
            lax.fori_loop(0, CPH // 2, step, None)
            pltpu.make_async_copy(feat_hbm.at[src_v.at[0]], rows0, sem0).wait()
        plsc.subcore_barrier()

        pltpu.sync_copy(acc_sh.at[pl.ds(sid * RPT, RPT)],
                        out_hbm.at[cid, pl.ds(sid * RPT, RPT)])
        if with_counts:
            pltpu.sync_copy(cnt_sh.at[pl.ds(sid * RPT, RPT)],
                            cnt_hbm.at[cid, pl.ds(sid * RPT, RPT)])

    return pl.kernel(body, out_type=out_type, mesh=mesh, scratch_types=scratch)


_sc_agg_counts = _make_sc_agg(True)
_sc_agg = _make_sc_agg(False)


R = 400
_f32 = jnp.float32


def _tc1_body(x_ref, p0_ref, p1_ref, c0_ref, c1_ref,
              wl1t_ref, bl1_ref, wr1t_ref, wlint_ref, blin_ref, out_ref):
    cnt = jnp.maximum(c0_ref[...] + c1_ref[...], 1.0)
    mean = (p0_ref[...] + p1_ref[...]) / cnt
    t = (jnp.dot(mean, wl1t_ref[...], preferred_element_type=_f32)
         + bl1_ref[...]
         + jnp.dot(x_ref[...], wr1t_ref[...], preferred_element_type=_f32))
    nrm = jnp.sqrt(jnp.sum(t * t, axis=1, keepdims=True))
    h1 = t / jnp.maximum(nrm, 1e-12)
    h = jnp.dot(h1, wlint_ref[...], preferred_element_type=_f32) + blin_ref[...]
    out_ref[...] = jnp.maximum(h, 0.0)


def _tc2_body(h_ref, p0_ref, p1_ref, c0_ref, c1_ref,
              wl2t_ref, bl2_ref, wr2t_ref, wlin2t_ref, blin2_ref,
              y_ref, p_ref):
    cnt = jnp.maximum(c0_ref[...] + c1_ref[...], 1.0)
    mean = (p0_ref[...] + p1_ref[...]) / cnt
    t = (jnp.dot(mean, wl2t_ref[...], preferred_element_type=_f32)
         + bl2_ref[...]
         + jnp.dot(h_ref[...], wr2t_ref[...], preferred_element_type=_f32))
    nrm = jnp.sqrt(jnp.sum(t * t, axis=1, keepdims=True))
    y = t / jnp.maximum(nrm, 1e-12)
    y_ref[...] = y
    logits = jnp.dot(y, wlin2t_ref[...], preferred_element_type=_f32) + blin2_ref[...]
    m = jnp.max(logits, axis=1, keepdims=True)
    e = jnp.exp(logits - m)
    p_ref[...] = e / jnp.sum(e, axis=1, keepdims=True)


_row_spec = pl.BlockSpec((R, D), lambda i: (i, 0))
_col_spec = pl.BlockSpec((R, 1), lambda i: (i, 0))
_w_spec = pl.BlockSpec((D, D), lambda i: (0, 0))
_b_spec = pl.BlockSpec((1, D), lambda i: (0, 0))

_tc1 = pl.pallas_call(
    _tc1_body,
    grid=(N // R,),
    in_specs=[_row_spec, _row_spec, _row_spec, _col_spec, _col_spec,
              _w_spec, _b_spec, _w_spec, _w_spec, _b_spec],
    out_specs=_row_spec,
    out_shape=jax.ShapeDtypeStruct((N, D), _f32),
)

_tc2 = pl.pallas_call(
    _tc2_body,
    grid=(N // R,),
    in_specs=[_row_spec, _row_spec, _row_spec, _col_spec, _col_spec,
              _w_spec, _b_spec, _w_spec, _w_spec, _b_spec],
    out_specs=[_row_spec, _row_spec],
    out_shape=[jax.ShapeDtypeStruct((N, D), _f32),
               jax.ShapeDtypeStruct((N, D), _f32)],
)


def kernel(x, edge_index, Wl1, bl1, Wr1, Wlin, blin, Wl2, bl2, Wr2, Wlin2, blin2):
    x = x.astype(jnp.float32)
    src = edge_index[0]
    dst = edge_index[1]
    pad = E_PAD - E
    spad = jnp.arange(pad, dtype=jnp.int32) * 37 % N
    srcb = jnp.concatenate([src, spad]).reshape(NW, CH_PER_W, CHUNK)
    dpad = N + (jnp.arange(pad, dtype=jnp.int32) % (ROWS_PAD - N))
    dstb = jnp.concatenate([dst, dpad]).reshape(NW, CH_PER_W, CHUNK)
    zf = jnp.zeros((ROWS_PAD, D), _f32)
    zc = jnp.zeros((ROWS_PAD,), _f32)

    agg1, cnt = _sc_agg_counts(x, srcb, dstb, zf, zc)
    c0 = cnt[0, :N, None]
    c1 = cnt[1, :N, None]
    h = _tc1(x, agg1[0, :N], agg1[1, :N], c0, c1,
             Wl1.T, bl1[None, :], Wr1.T, Wlin.T, blin[None, :])

    (agg2,) = _sc_agg(h, srcb, dstb, zf, zc)
    wlin2t = jnp.zeros((D, D), _f32).at[:, :2].set(Wlin2.T)
    blin2p = jnp.full((1, D), -1e30, _f32).at[0, :2].set(blin2)
    y, p_pad = _tc2(h, agg2[0, :N], agg2[1, :N], c0, c1,
                    Wl2.T, bl2[None, :], Wr2.T, wlin2t, blin2p)
    return (p_pad[:, :2], y)


# speedup vs baseline: 3.6753x; 1.0492x over previous
import functools

import jax
import jax.numpy as jnp
from jax import lax
from jax.experimental import pallas as pl
from jax.experimental.pallas import tpu as pltpu
from jax.experimental.pallas import tpu_sc as plsc

N = 10000
E = 320000
D = 128

NC = 2
NS = 16
NW = NC * NS
CHUNK = 128
CH_PER_W = 2 * (-(-E // (NW * CHUNK * 2)))
E_PAD = NW * CH_PER_W * CHUNK
HALVES = 2
CPH = CH_PER_W // HALVES
ROWS_PAD = 10240
RPT = ROWS_PAD // NS


def _make_sc_agg(with_counts: bool):
    mesh = plsc.VectorSubcoreMesh(core_axis_name="c", subcore_axis_name="s",
                                  num_cores=NC, num_subcores=NS)
    out_type = [jax.ShapeDtypeStruct((NC, ROWS_PAD, D), jnp.float32)]
    scratch = [
        pltpu.VMEM((CPH, CHUNK), jnp.int32),
        pltpu.VMEM((CPH, CHUNK), jnp.int32),
        pltpu.VMEM((CHUNK, D), jnp.float32),
        pltpu.VMEM((CHUNK, D), jnp.float32),
        pltpu.VMEM_SHARED((ROWS_PAD, D), jnp.float32),
        pltpu.SemaphoreType.DMA,
        pltpu.SemaphoreType.DMA,
    ]
    if with_counts:
        out_type.append(jax.ShapeDtypeStruct((NC, ROWS_PAD), jnp.float32))
        scratch += [
            pltpu.VMEM((CHUNK,), jnp.float32),
            pltpu.VMEM_SHARED((ROWS_PAD,), jnp.float32),
        ]

    def body(feat_hbm, src_hbm, dst_hbm, zf_hbm, zc_hbm, *rest):
        if with_counts:
            (out_hbm, cnt_hbm, src_v, dst_v, rows0, rows1, acc_sh, sem0, sem1,
             ones_v, cnt_sh) = rest
        else:
            out_hbm, src_v, dst_v, rows0, rows1, acc_sh, sem0, sem1 = rest
        cid = lax.axis_index("c")
        sid = lax.axis_index("s")
        wid = cid * NS + sid

        pltpu.sync_copy(zf_hbm.at[pl.ds(sid * RPT, RPT)],
                        acc_sh.at[pl.ds(sid * RPT, RPT)])
        if with_counts:
            pltpu.sync_copy(zc_hbm.at[pl.ds(sid * RPT, RPT)],
                            cnt_sh.at[pl.ds(sid * RPT, RPT)])
            for k in range(CHUNK // 16):
                ones_v[pl.ds(k * 16, 16)] = jnp.ones((16,), jnp.float32)
        plsc.subcore_barrier()

        def step(i, carry):
            j = 2 * i
            pltpu.async_copy(feat_hbm.at[src_v.at[j + 1]], rows1, sem1)
            pltpu.make_async_copy(feat_hbm.at[src_v.at[j]], rows0, sem0).wait()
            pltpu.sync_copy(rows0, acc_sh.at[dst_v.at[j]], add=True)
            if with_counts:
                pltpu.sync_copy(ones_v, cnt_sh.at[dst_v.at[j]], add=True)
            nxt = jnp.minimum(j + 2, CPH - 2)
            pltpu.async_copy(feat_hbm.at[src_v.at[nxt]], rows0, sem0)
            pltpu.make_async_copy(feat_hbm.at[src_v.at[j + 1]], rows1, sem1).wait()
            pltpu.sync_copy(rows1, acc_sh.at[dst_v.at[j + 1]], add=True)
            if with_counts:
                pltpu.sync_copy(ones_v, cnt_sh.at[dst_v.at[j + 1]], add=True)
            return carry

        for half in range(HALVES):
            pltpu.sync_copy(src_hbm.at[wid, pl.ds(half * CPH, CPH)], src_v)
            pltpu.sync_copy(dst_hbm.at[wid, pl.ds(half * CPH, CPH)], dst_v)
            pltpu.async_copy(feat_hbm.at[src_v.at[0]], rows0, sem0)
            lax.fori_loop(0, CPH // 2, step, None)
            pltpu.make_async_copy(feat_hbm.at[src_v.at[0]], rows0, sem0).wait()
        plsc.subcore_barrier()

        pltpu.sync_copy(acc_sh.at[pl.ds(sid * RPT, RPT)],
                        out_hbm.at[cid, pl.ds(sid * RPT, RPT)])
        if with_counts:
            pltpu.sync_copy(cnt_sh.at[pl.ds(sid * RPT, RPT)],
                            cnt_hbm.at[cid, pl.ds(sid * RPT, RPT)])

    return pl.kernel(body, out_type=out_type, mesh=mesh, scratch_types=scratch)


_sc_agg_counts = _make_sc_agg(True)
_sc_agg = _make_sc_agg(False)


R = 400
_f32 = jnp.float32


def _tc1_body(x_ref, p0_ref, p1_ref, c0_ref, c1_ref,
              wl1t_ref, bl1_ref, wr1t_ref, wlint_ref, blin_ref, out_ref):
    cnt = jnp.maximum(c0_ref[...] + c1_ref[...], 1.0)
    mean = (p0_ref[0] + p1_ref[0]) / cnt
    t = (jnp.dot(mean, wl1t_ref[...], preferred_element_type=_f32)
         + bl1_ref[...]
         + jnp.dot(x_ref[...], wr1t_ref[...], preferred_element_type=_f32))
    nrm = jnp.sqrt(jnp.sum(t * t, axis=1, keepdims=True))
    h1 = t / jnp.maximum(nrm, 1e-12)
    h = jnp.dot(h1, wlint_ref[...], preferred_element_type=_f32) + blin_ref[...]
    out_ref[...] = jnp.maximum(h, 0.0)


def _tc2_body(h_ref, p0_ref, p1_ref, c0_ref, c1_ref,
              wl2t_ref, bl2_ref, wr2t_ref, wlin2t_ref, blin2_ref,
              y_ref, p_ref):
    cnt = jnp.maximum(c0_ref[...] + c1_ref[...], 1.0)
    mean = (p0_ref[0] + p1_ref[0]) / cnt
    t = (jnp.dot(mean, wl2t_ref[...], preferred_element_type=_f32)
         + bl2_ref[...]
         + jnp.dot(h_ref[...], wr2t_ref[...], preferred_element_type=_f32))
    nrm = jnp.sqrt(jnp.sum(t * t, axis=1, keepdims=True))
    y = t / jnp.maximum(nrm, 1e-12)
    y_ref[...] = y
    logits = jnp.dot(y, wlin2t_ref[...], preferred_element_type=_f32) + blin2_ref[...]
    m = jnp.max(logits, axis=1, keepdims=True)
    e = jnp.exp(logits - m)
    p_ref[...] = e / jnp.sum(e, axis=1, keepdims=True)


_row_spec = pl.BlockSpec((R, D), lambda i: (i, 0))
_agg0_spec = pl.BlockSpec((1, R, D), lambda i: (0, i, 0))
_agg1_spec = pl.BlockSpec((1, R, D), lambda i: (1, i, 0))
_col_spec = pl.BlockSpec((R, 1), lambda i: (i, 0))
_w_spec = pl.BlockSpec((D, D), lambda i: (0, 0))
_b_spec = pl.BlockSpec((1, D), lambda i: (0, 0))

_tc1 = pl.pallas_call(
    _tc1_body,
    grid=(N // R,),
    in_specs=[_row_spec, _agg0_spec, _agg1_spec, _col_spec, _col_spec,
              _w_spec, _b_spec, _w_spec, _w_spec, _b_spec],
    out_specs=_row_spec,
    out_shape=jax.ShapeDtypeStruct((N, D), _f32),
)

_tc2 = pl.pallas_call(
    _tc2_body,
    grid=(N // R,),
    in_specs=[_row_spec, _agg0_spec, _agg1_spec, _col_spec, _col_spec,
              _w_spec, _b_spec, _w_spec, _w_spec, _b_spec],
    out_specs=[_row_spec, _row_spec],
    out_shape=[jax.ShapeDtypeStruct((N, D), _f32),
               jax.ShapeDtypeStruct((N, D), _f32)],
)


def kernel(x, edge_index, Wl1, bl1, Wr1, Wlin, blin, Wl2, bl2, Wr2, Wlin2, blin2):
    x = x.astype(jnp.float32)
    src = edge_index[0]
    dst = edge_index[1]
    pad = E_PAD - E
    spad = jnp.arange(pad, dtype=jnp.int32) * 37 % N
    srcb = jnp.concatenate([src, spad]).reshape(NW, CH_PER_W, CHUNK)
    dpad = N + (jnp.arange(pad, dtype=jnp.int32) % (ROWS_PAD - N))
    dstb = jnp.concatenate([dst, dpad]).reshape(NW, CH_PER_W, CHUNK)
    zf = jnp.zeros((ROWS_PAD, D), _f32)
    zc = jnp.zeros((ROWS_PAD,), _f32)

    agg1, cnt = _sc_agg_counts(x, srcb, dstb, zf, zc)
    c0 = cnt[0, :N, None]
    c1 = cnt[1, :N, None]
    h = _tc1(x, agg1, agg1, c0, c1,
             Wl1.T, bl1[None, :], Wr1.T, Wlin.T, blin[None, :])

    (agg2,) = _sc_agg(h, srcb, dstb, zf, zc)
    wlin2t = jnp.zeros((D, D), _f32).at[:, :2].set(Wlin2.T)
    blin2p = jnp.full((1, D), -1e30, _f32).at[0, :2].set(blin2)
    y, p_pad = _tc2(h, agg2, agg2, c0, c1,
                    Wl2.T, bl2[None, :], Wr2.T, wlin2t, blin2p)
    return (p_pad[:, :2], y)


# speedup vs baseline: 3.8756x; 1.0545x over previous
import functools

import jax
import jax.numpy as jnp
from jax import lax
from jax.experimental import pallas as pl
from jax.experimental.pallas import tpu as pltpu
from jax.experimental.pallas import tpu_sc as plsc

N = 10000
E = 320000
D = 128

NC = 2
NS = 16
NW = NC * NS
CHUNK = 128
CH_PER_W = 2 * (-(-E // (NW * CHUNK * 2)))
E_PAD = NW * CH_PER_W * CHUNK
HALVES = 2
CPH = CH_PER_W // HALVES
ROWS_PAD = 10240
RPT = ROWS_PAD // NS


def _make_sc_agg(with_counts: bool):
    mesh = plsc.VectorSubcoreMesh(core_axis_name="c", subcore_axis_name="s",
                                  num_cores=NC, num_subcores=NS)
    out_type = [jax.ShapeDtypeStruct((NC, ROWS_PAD, D), jnp.float32)]
    scratch = [
        pltpu.VMEM((CPH, CHUNK), jnp.int32),
        pltpu.VMEM((CPH, CHUNK), jnp.int32),
        pltpu.VMEM((CHUNK, D), jnp.float32),
        pltpu.VMEM((CHUNK, D), jnp.float32),
        pltpu.VMEM_SHARED((ROWS_PAD, D), jnp.float32),
        pltpu.SemaphoreType.DMA,
        pltpu.SemaphoreType.DMA,
    ]
    if with_counts:
        out_type.append(jax.ShapeDtypeStruct((NC, ROWS_PAD), jnp.float32))
        scratch += [
            pltpu.VMEM((CHUNK,), jnp.float32),
            pltpu.VMEM_SHARED((ROWS_PAD,), jnp.float32),
        ]

    def body(feat_hbm, src_hbm, dst_hbm, zf_hbm, zc_hbm, *rest):
        if with_counts:
            (out_hbm, cnt_hbm, src_v, dst_v, rows0, rows1, acc_sh, sem0, sem1,
             ones_v, cnt_sh) = rest
        else:
            out_hbm, src_v, dst_v, rows0, rows1, acc_sh, sem0, sem1 = rest
        cid = lax.axis_index("c")
        sid = lax.axis_index("s")
        wid = cid * NS + sid

        pltpu.sync_copy(zf_hbm.at[cid, pl.ds(sid * RPT, RPT)],
                        acc_sh.at[pl.ds(sid * RPT, RPT)])
        if with_counts:
            pltpu.sync_copy(zc_hbm.at[cid, pl.ds(sid * RPT, RPT)],
                            cnt_sh.at[pl.ds(sid * RPT, RPT)])
            for k in range(CHUNK // 16):
                ones_v[pl.ds(k * 16, 16)] = jnp.ones((16,), jnp.float32)
        plsc.subcore_barrier()

        def step(i, carry):
            j = 2 * i
            pltpu.async_copy(feat_hbm.at[src_v.at[j + 1]], rows1, sem1)
            pltpu.make_async_copy(feat_hbm.at[src_v.at[j]], rows0, sem0).wait()
            pltpu.sync_copy(rows0, acc_sh.at[dst_v.at[j]], add=True)
            if with_counts:
                pltpu.sync_copy(ones_v, cnt_sh.at[dst_v.at[j]], add=True)
            nxt = jnp.minimum(j + 2, CPH - 2)
            pltpu.async_copy(feat_hbm.at[src_v.at[nxt]], rows0, sem0)
            pltpu.make_async_copy(feat_hbm.at[src_v.at[j + 1]], rows1, sem1).wait()
            pltpu.sync_copy(rows1, acc_sh.at[dst_v.at[j + 1]], add=True)
            if with_counts:
                pltpu.sync_copy(ones_v, cnt_sh.at[dst_v.at[j + 1]], add=True)
            return carry

        for half in range(HALVES):
            pltpu.sync_copy(src_hbm.at[wid, pl.ds(half * CPH, CPH)], src_v)
            pltpu.sync_copy(dst_hbm.at[wid, pl.ds(half * CPH, CPH)], dst_v)
            pltpu.async_copy(feat_hbm.at[src_v.at[0]], rows0, sem0)
            lax.fori_loop(0, CPH // 2, step, None)
            pltpu.make_async_copy(feat_hbm.at[src_v.at[0]], rows0, sem0).wait()
        plsc.subcore_barrier()

        pltpu.sync_copy(acc_sh.at[pl.ds(sid * RPT, RPT)],
                        out_hbm.at[cid, pl.ds(sid * RPT, RPT)])
        if with_counts:
            pltpu.sync_copy(cnt_sh.at[pl.ds(sid * RPT, RPT)],
                            cnt_hbm.at[cid, pl.ds(sid * RPT, RPT)])

    return pl.kernel(body, out_type=out_type, mesh=mesh, scratch_types=scratch)


_sc_agg_counts = _make_sc_agg(True)
_sc_agg = _make_sc_agg(False)


R = 1000
_f32 = jnp.float32


def _tc1_body(x_ref, p0_ref, p1_ref, c0_ref, c1_ref,
              wl1t_ref, bl1_ref, wr1t_ref, wlint_ref, blin_ref, out_ref):
    cnt = jnp.maximum(c0_ref[...] + c1_ref[...], 1.0)
    mean = (p0_ref[0] + p1_ref[0]) / cnt
    t = (jnp.dot(mean, wl1t_ref[...], preferred_element_type=_f32)
         + bl1_ref[...]
         + jnp.dot(x_ref[...], wr1t_ref[...], preferred_element_type=_f32))
    nrm = jnp.sqrt(jnp.sum(t * t, axis=1, keepdims=True))
    h1 = t / jnp.maximum(nrm, 1e-12)
    h = jnp.dot(h1, wlint_ref[...], preferred_element_type=_f32) + blin_ref[...]
    out_ref[...] = jnp.maximum(h, 0.0)


def _tc2_body(h_ref, p0_ref, p1_ref, c0_ref, c1_ref,
              wl2t_ref, bl2_ref, wr2t_ref, wlin2t_ref, blin2_ref,
              y_ref, p_ref):
    cnt = jnp.maximum(c0_ref[...] + c1_ref[...], 1.0)
    mean = (p0_ref[0] + p1_ref[0]) / cnt
    t = (jnp.dot(mean, wl2t_ref[...], preferred_element_type=_f32)
         + bl2_ref[...]
         + jnp.dot(h_ref[...], wr2t_ref[...], preferred_element_type=_f32))
    nrm = jnp.sqrt(jnp.sum(t * t, axis=1, keepdims=True))
    y = t / jnp.maximum(nrm, 1e-12)
    y_ref[...] = y
    logits = jnp.dot(y, wlin2t_ref[...], preferred_element_type=_f32) + blin2_ref[...]
    m = jnp.max(logits, axis=1, keepdims=True)
    e = jnp.exp(logits - m)
    p_ref[...] = e / jnp.sum(e, axis=1, keepdims=True)


_row_spec = pl.BlockSpec((R, D), lambda i: (i, 0))
_agg0_spec = pl.BlockSpec((1, R, D), lambda i: (0, i, 0))
_agg1_spec = pl.BlockSpec((1, R, D), lambda i: (1, i, 0))
_col_spec = pl.BlockSpec((R, 1), lambda i: (i, 0))
_w_spec = pl.BlockSpec((D, D), lambda i: (0, 0))
_b_spec = pl.BlockSpec((1, D), lambda i: (0, 0))

_tc1 = pl.pallas_call(
    _tc1_body,
    grid=(N // R,),
    in_specs=[_row_spec, _agg0_spec, _agg1_spec, _col_spec, _col_spec,
              _w_spec, _b_spec, _w_spec, _w_spec, _b_spec],
    out_specs=_row_spec,
    out_shape=jax.ShapeDtypeStruct((N, D), _f32),
)

_tc2 = pl.pallas_call(
    _tc2_body,
    grid=(N // R,),
    in_specs=[_row_spec, _agg0_spec, _agg1_spec, _col_spec, _col_spec,
              _w_spec, _b_spec, _w_spec, _w_spec, _b_spec],
    out_specs=[_row_spec, _row_spec],
    out_shape=[jax.ShapeDtypeStruct((N, D), _f32),
               jax.ShapeDtypeStruct((N, D), _f32)],
)


def kernel(x, edge_index, Wl1, bl1, Wr1, Wlin, blin, Wl2, bl2, Wr2, Wlin2, blin2):
    x = x.astype(jnp.float32)
    src = edge_index[0]
    dst = edge_index[1]
    pad = E_PAD - E
    spad = jnp.arange(pad, dtype=jnp.int32) * 37 % N
    srcb = jnp.concatenate([src, spad]).reshape(NW, CH_PER_W, CHUNK)
    dpad = N + (jnp.arange(pad, dtype=jnp.int32) % (ROWS_PAD - N))
    dstb = jnp.concatenate([dst, dpad]).reshape(NW, CH_PER_W, CHUNK)
    zf = jnp.zeros((NC, ROWS_PAD, D), _f32)
    zc = jnp.zeros((NC, ROWS_PAD), _f32)

    agg1, cnt = _sc_agg_counts(x, srcb, dstb, zf, zc)
    c0 = cnt[0, :N, None]
    c1 = cnt[1, :N, None]
    h = _tc1(x, agg1, agg1, c0, c1,
             Wl1.T, bl1[None, :], Wr1.T, Wlin.T, blin[None, :])

    (agg2,) = _sc_agg(h, srcb, dstb, zf, zc)
    wlin2t = jnp.zeros((D, D), _f32).at[:, :2].set(Wlin2.T)
    blin2p = jnp.full((1, D), -1e30, _f32).at[0, :2].set(blin2)
    y, p_pad = _tc2(h, agg2, agg2, c0, c1,
                    Wl2.T, bl2[None, :], Wr2.T, wlin2t, blin2p)
    return (p_pad[:, :2], y)


# speedup vs baseline: 3.8765x; 1.0002x over previous
import functools

import jax
import jax.numpy as jnp
from jax import lax
from jax.experimental import pallas as pl
from jax.experimental.pallas import tpu as pltpu
from jax.experimental.pallas import tpu_sc as plsc

N = 10000
E = 320000
D = 128

NC = 2
NS = 16
NW = NC * NS
CHUNK = 128
CH_PER_W = 2 * (-(-E // (NW * CHUNK * 2)))
E_PAD = NW * CH_PER_W * CHUNK
HALVES = 2
CPH = CH_PER_W // HALVES
ROWS_PAD = 10240
RPT = ROWS_PAD // NS


def _make_sc_agg(with_counts: bool):
    mesh = plsc.VectorSubcoreMesh(core_axis_name="c", subcore_axis_name="s",
                                  num_cores=NC, num_subcores=NS)
    out_type = [jax.ShapeDtypeStruct((NC, ROWS_PAD, D), jnp.float32)]
    scratch = [
        pltpu.VMEM((CPH, CHUNK), jnp.int32),
        pltpu.VMEM((CPH, CHUNK), jnp.int32),
        pltpu.VMEM((CHUNK, D), jnp.float32),
        pltpu.VMEM((CHUNK, D), jnp.float32),
        pltpu.VMEM_SHARED((ROWS_PAD, D), jnp.float32),
        pltpu.SemaphoreType.DMA,
        pltpu.SemaphoreType.DMA,
    ]
    if with_counts:
        out_type.append(jax.ShapeDtypeStruct((NC, ROWS_PAD), jnp.float32))
        scratch += [
            pltpu.VMEM((CHUNK,), jnp.float32),
            pltpu.VMEM_SHARED((ROWS_PAD,), jnp.float32),
        ]

    def body(feat_hbm, src_hbm, dst_hbm, zf_hbm, zc_hbm, *rest):
        if with_counts:
            (out_hbm, cnt_hbm, src_v, dst_v, rows0, rows1, acc_sh, sem0, sem1,
             ones_v, cnt_sh) = rest
        else:
            out_hbm, src_v, dst_v, rows0, rows1, acc_sh, sem0, sem1 = rest
        cid = lax.axis_index("c")
        sid = lax.axis_index("s")
        wid = cid * NS + sid

        pltpu.sync_copy(zf_hbm.at[cid, pl.ds(sid * RPT, RPT)],
                        acc_sh.at[pl.ds(sid * RPT, RPT)])
        if with_counts:
            pltpu.sync_copy(zc_hbm.at[cid, pl.ds(sid * RPT, RPT)],
                            cnt_sh.at[pl.ds(sid * RPT, RPT)])
            for k in range(CHUNK // 16):
                ones_v[pl.ds(k * 16, 16)] = jnp.ones((16,), jnp.float32)
        plsc.subcore_barrier()

        def step(i, carry):
            j = 2 * i
            pltpu.async_copy(feat_hbm.at[src_v.at[j + 1]], rows1, sem1)
            pltpu.make_async_copy(feat_hbm.at[src_v.at[j]], rows0, sem0).wait()
            pltpu.sync_copy(rows0, acc_sh.at[dst_v.at[j]], add=True)
            if with_counts:
                pltpu.sync_copy(ones_v, cnt_sh.at[dst_v.at[j]], add=True)
            nxt = jnp.minimum(j + 2, CPH - 2)
            pltpu.async_copy(feat_hbm.at[src_v.at[nxt]], rows0, sem0)
            pltpu.make_async_copy(feat_hbm.at[src_v.at[j + 1]], rows1, sem1).wait()
            pltpu.sync_copy(rows1, acc_sh.at[dst_v.at[j + 1]], add=True)
            if with_counts:
                pltpu.sync_copy(ones_v, cnt_sh.at[dst_v.at[j + 1]], add=True)
            return carry

        for half in range(HALVES):
            pltpu.sync_copy(src_hbm.at[wid, pl.ds(half * CPH, CPH)], src_v)
            pltpu.sync_copy(dst_hbm.at[wid, pl.ds(half * CPH, CPH)], dst_v)
            pltpu.async_copy(feat_hbm.at[src_v.at[0]], rows0, sem0)
            lax.fori_loop(0, CPH // 2, step, None)
            pltpu.make_async_copy(feat_hbm.at[src_v.at[0]], rows0, sem0).wait()
        plsc.subcore_barrier()

        pltpu.sync_copy(acc_sh.at[pl.ds(sid * RPT, RPT)],
                        out_hbm.at[cid, pl.ds(sid * RPT, RPT)])
        if with_counts:
            pltpu.sync_copy(cnt_sh.at[pl.ds(sid * RPT, RPT)],
                            cnt_hbm.at[cid, pl.ds(sid * RPT, RPT)])

    return pl.kernel(body, out_type=out_type, mesh=mesh, scratch_types=scratch)


_sc_agg_counts = _make_sc_agg(True)
_sc_agg = _make_sc_agg(False)


R = 1000
_f32 = jnp.float32


def _tc1_body(x_ref, p0_ref, p1_ref, c0_ref, c1_ref,
              wl1t_ref, bl1_ref, wr1t_ref, wlint_ref, blin_ref, out_ref):
    cnt = jnp.maximum(c0_ref[...] + c1_ref[...], 1.0)
    mean = (p0_ref[0] + p1_ref[0]) / cnt
    t = (jnp.dot(mean, wl1t_ref[...], preferred_element_type=_f32)
         + bl1_ref[...]
         + jnp.dot(x_ref[...], wr1t_ref[...], preferred_element_type=_f32))
    nrm = jnp.sqrt(jnp.sum(t * t, axis=1, keepdims=True))
    h1 = t / jnp.maximum(nrm, 1e-12)
    h = jnp.dot(h1, wlint_ref[...], preferred_element_type=_f32) + blin_ref[...]
    out_ref[...] = jnp.maximum(h, 0.0)


def _tc2_body(h_ref, p0_ref, p1_ref, c0_ref, c1_ref,
              wl2t_ref, bl2_ref, wr2t_ref, wlin2t_ref, blin2_ref,
              y_ref, p_ref):
    cnt = jnp.maximum(c0_ref[...] + c1_ref[...], 1.0)
    mean = (p0_ref[0] + p1_ref[0]) / cnt
    t = (jnp.dot(mean, wl2t_ref[...], preferred_element_type=_f32)
         + bl2_ref[...]
         + jnp.dot(h_ref[...], wr2t_ref[...], preferred_element_type=_f32))
    nrm = jnp.sqrt(jnp.sum(t * t, axis=1, keepdims=True))
    y = t / jnp.maximum(nrm, 1e-12)
    y_ref[...] = y
    logits = jnp.dot(y, wlin2t_ref[...], preferred_element_type=_f32) + blin2_ref[...]
    m = jnp.max(logits, axis=1, keepdims=True)
    e = jnp.exp(logits - m)
    p_ref[...] = (e / jnp.sum(e, axis=1, keepdims=True))[:, :2]


_row_spec = pl.BlockSpec((R, D), lambda i: (i, 0))
_agg0_spec = pl.BlockSpec((1, R, D), lambda i: (0, i, 0))
_agg1_spec = pl.BlockSpec((1, R, D), lambda i: (1, i, 0))
_col_spec = pl.BlockSpec((R, 1), lambda i: (i, 0))
_w_spec = pl.BlockSpec((D, D), lambda i: (0, 0))
_b_spec = pl.BlockSpec((1, D), lambda i: (0, 0))

_tc1 = pl.pallas_call(
    _tc1_body,
    grid=(N // R,),
    in_specs=[_row_spec, _agg0_spec, _agg1_spec, _col_spec, _col_spec,
              _w_spec, _b_spec, _w_spec, _w_spec, _b_spec],
    out_specs=_row_spec,
    out_shape=jax.ShapeDtypeStruct((N, D), _f32),
)

_tc2 = pl.pallas_call(
    _tc2_body,
    grid=(N // R,),
    in_specs=[_row_spec, _agg0_spec, _agg1_spec, _col_spec, _col_spec,
              _w_spec, _b_spec, _w_spec, _w_spec, _b_spec],
    out_specs=[_row_spec, pl.BlockSpec((R, 2), lambda i: (i, 0))],
    out_shape=[jax.ShapeDtypeStruct((N, D), _f32),
               jax.ShapeDtypeStruct((N, 2), _f32)],
)


def kernel(x, edge_index, Wl1, bl1, Wr1, Wlin, blin, Wl2, bl2, Wr2, Wlin2, blin2):
    x = x.astype(jnp.float32)
    src = edge_index[0]
    dst = edge_index[1]
    pad = E_PAD - E
    spad = jnp.arange(pad, dtype=jnp.int32) * 37 % N
    srcb = jnp.concatenate([src, spad]).reshape(NW, CH_PER_W, CHUNK)
    dpad = N + (jnp.arange(pad, dtype=jnp.int32) % (ROWS_PAD - N))
    dstb = jnp.concatenate([dst, dpad]).reshape(NW, CH_PER_W, CHUNK)
    zf = jnp.zeros((NC, ROWS_PAD, D), _f32)
    zc = jnp.zeros((NC, ROWS_PAD), _f32)

    agg1, cnt = _sc_agg_counts(x, srcb, dstb, zf, zc)
    c0 = cnt[0, :N, None]
    c1 = cnt[1, :N, None]
    h = _tc1(x, agg1, agg1, c0, c1,
             Wl1.T, bl1[None, :], Wr1.T, Wlin.T, blin[None, :])

    (agg2,) = _sc_agg(h, srcb, dstb, zf, zc)
    wlin2t = jnp.zeros((D, D), _f32).at[:, :2].set(Wlin2.T)
    blin2p = jnp.full((1, D), -1e30, _f32).at[0, :2].set(blin2)
    y, p = _tc2(h, agg2, agg2, c0, c1,
                Wl2.T, bl2[None, :], Wr2.T, wlin2t, blin2p)
    return (p, y)


# speedup vs baseline: 3.8781x; 1.0004x over previous
import jax
import jax.numpy as jnp
from jax import lax
from jax.experimental import pallas as pl
from jax.experimental.pallas import tpu as pltpu
from jax.experimental.pallas import tpu_sc as plsc

N = 10000
E = 320000
D = 128

NC = 2
NS = 16
NW = NC * NS
CHUNK = 128
CH_PER_W = 2 * (-(-E // (NW * CHUNK * 2)))
E_PAD = NW * CH_PER_W * CHUNK
HALVES = 2
CPH = CH_PER_W // HALVES
ROWS_PAD = 10240
RPT = ROWS_PAD // NS


def _make_sc_agg(with_counts: bool):
    mesh = plsc.VectorSubcoreMesh(core_axis_name="c", subcore_axis_name="s",
                                  num_cores=NC, num_subcores=NS)
    out_type = [jax.ShapeDtypeStruct((NC, ROWS_PAD, D), jnp.float32)]
    scratch = [
        pltpu.VMEM((CPH, CHUNK), jnp.int32),
        pltpu.VMEM((CPH, CHUNK), jnp.int32),
        pltpu.VMEM((CHUNK, D), jnp.float32),
        pltpu.VMEM((CHUNK, D), jnp.float32),
        pltpu.VMEM_SHARED((ROWS_PAD, D), jnp.float32),
        pltpu.SemaphoreType.DMA,
        pltpu.SemaphoreType.DMA,
    ]
    if with_counts:
        out_type.append(jax.ShapeDtypeStruct((NC, ROWS_PAD), jnp.float32))
        scratch += [
            pltpu.VMEM((CHUNK,), jnp.float32),
            pltpu.VMEM_SHARED((ROWS_PAD,), jnp.float32),
        ]

    def body(feat_hbm, src_hbm, dst_hbm, zf_hbm, zc_hbm, *rest):
        if with_counts:
            (out_hbm, cnt_hbm, src_v, dst_v, rows0, rows1, acc_sh, sem0, sem1,
             ones_v, cnt_sh) = rest
        else:
            out_hbm, src_v, dst_v, rows0, rows1, acc_sh, sem0, sem1 = rest
        cid = lax.axis_index("c")
        sid = lax.axis_index("s")
        wid = cid * NS + sid

        pltpu.sync_copy(zf_hbm.at[cid, pl.ds(sid * RPT, RPT)],
                        acc_sh.at[pl.ds(sid * RPT, RPT)])
        if with_counts:
            pltpu.sync_copy(zc_hbm.at[cid, pl.ds(sid * RPT, RPT)],
                            cnt_sh.at[pl.ds(sid * RPT, RPT)])
            for k in range(CHUNK // 16):
                ones_v[pl.ds(k * 16, 16)] = jnp.ones((16,), jnp.float32)
        plsc.subcore_barrier()

        def step(i, carry):
            j = 2 * i
            pltpu.async_copy(feat_hbm.at[src_v.at[j + 1]], rows1, sem1)
            pltpu.make_async_copy(feat_hbm.at[src_v.at[j]], rows0, sem0).wait()
            pltpu.sync_copy(rows0, acc_sh.at[dst_v.at[j]], add=True)
            if with_counts:
                pltpu.sync_copy(ones_v, cnt_sh.at[dst_v.at[j]], add=True)
            nxt = jnp.minimum(j + 2, CPH - 2)
            pltpu.async_copy(feat_hbm.at[src_v.at[nxt]], rows0, sem0)
            pltpu.make_async_copy(feat_hbm.at[src_v.at[j + 1]], rows1, sem1).wait()
            pltpu.sync_copy(rows1, acc_sh.at[dst_v.at[j + 1]], add=True)
            if with_counts:
                pltpu.sync_copy(ones_v, cnt_sh.at[dst_v.at[j + 1]], add=True)
            return carry

        for half in range(HALVES):
            pltpu.sync_copy(src_hbm.at[wid, pl.ds(half * CPH, CPH)], src_v)
            pltpu.sync_copy(dst_hbm.at[wid, pl.ds(half * CPH, CPH)], dst_v)
            pltpu.async_copy(feat_hbm.at[src_v.at[0]], rows0, sem0)
            lax.fori_loop(0, CPH // 2, step, None)
            pltpu.make_async_copy(feat_hbm.at[src_v.at[0]], rows0, sem0).wait()
        plsc.subcore_barrier()

        pltpu.sync_copy(acc_sh.at[pl.ds(sid * RPT, RPT)],
                        out_hbm.at[cid, pl.ds(sid * RPT, RPT)])
        if with_counts:
            pltpu.sync_copy(cnt_sh.at[pl.ds(sid * RPT, RPT)],
                            cnt_hbm.at[cid, pl.ds(sid * RPT, RPT)])

    return pl.kernel(body, out_type=out_type, mesh=mesh, scratch_types=scratch)


_sc_agg_counts = _make_sc_agg(True)
_sc_agg = _make_sc_agg(False)


R = 1000
_f32 = jnp.float32


def _tc1_body(x_ref, p0_ref, p1_ref, c0_ref, c1_ref,
              wl1t_ref, bl1_ref, wr1t_ref, wlint_ref, blin_ref, out_ref):
    cnt = jnp.maximum(c0_ref[...] + c1_ref[...], 1.0)
    mean = (p0_ref[0] + p1_ref[0]) / cnt
    t = (jnp.dot(mean, wl1t_ref[...], preferred_element_type=_f32)
         + bl1_ref[...]
         + jnp.dot(x_ref[...], wr1t_ref[...], preferred_element_type=_f32))
    nrm = jnp.sqrt(jnp.sum(t * t, axis=1, keepdims=True))
    h1 = t / jnp.maximum(nrm, 1e-12)
    h = jnp.dot(h1, wlint_ref[...], preferred_element_type=_f32) + blin_ref[...]
    out_ref[...] = jnp.maximum(h, 0.0)


def _tc2_body(h_ref, p0_ref, p1_ref, c0_ref, c1_ref,
              wl2t_ref, bl2_ref, wr2t_ref, wlin2t_ref, blin2_ref,
              y_ref, p_ref):
    cnt = jnp.maximum(c0_ref[...] + c1_ref[...], 1.0)
    mean = (p0_ref[0] + p1_ref[0]) / cnt
    t = (jnp.dot(mean, wl2t_ref[...], preferred_element_type=_f32)
         + bl2_ref[...]
         + jnp.dot(h_ref[...], wr2t_ref[...], preferred_element_type=_f32))
    nrm = jnp.sqrt(jnp.sum(t * t, axis=1, keepdims=True))
    y = t / jnp.maximum(nrm, 1e-12)
    y_ref[...] = y
    logits = jnp.dot(y, wlin2t_ref[...], preferred_element_type=_f32) + blin2_ref[...]
    m = jnp.max(logits, axis=1, keepdims=True)
    e = jnp.exp(logits - m)
    p_ref[...] = (e / jnp.sum(e, axis=1, keepdims=True))[:, :2]


_row_spec = pl.BlockSpec((R, D), lambda i: (i, 0))
_agg0_spec = pl.BlockSpec((1, R, D), lambda i: (0, i, 0))
_agg1_spec = pl.BlockSpec((1, R, D), lambda i: (1, i, 0))
_col_spec = pl.BlockSpec((R, 1), lambda i: (i, 0))
_w_spec = pl.BlockSpec((D, D), lambda i: (0, 0))
_b_spec = pl.BlockSpec((1, D), lambda i: (0, 0))

_tc1 = pl.pallas_call(
    _tc1_body,
    grid=(N // R,),
    in_specs=[_row_spec, _agg0_spec, _agg1_spec, _col_spec, _col_spec,
              _w_spec, _b_spec, _w_spec, _w_spec, _b_spec],
    out_specs=_row_spec,
    out_shape=jax.ShapeDtypeStruct((N, D), _f32),
)

_tc2 = pl.pallas_call(
    _tc2_body,
    grid=(N // R,),
    in_specs=[_row_spec, _agg0_spec, _agg1_spec, _col_spec, _col_spec,
              _w_spec, _b_spec, _w_spec, _w_spec, _b_spec],
    out_specs=[_row_spec, pl.BlockSpec((R, 2), lambda i: (i, 0))],
    out_shape=[jax.ShapeDtypeStruct((N, D), _f32),
               jax.ShapeDtypeStruct((N, 2), _f32)],
)


def kernel(x, edge_index, Wl1, bl1, Wr1, Wlin, blin, Wl2, bl2, Wr2, Wlin2, blin2):
    x = x.astype(jnp.float32)
    src = edge_index[0]
    dst = edge_index[1]
    pad = E_PAD - E
    spad = jnp.arange(pad, dtype=jnp.int32) * 37 % N
    srcb = jnp.concatenate([src, spad]).reshape(NW, CH_PER_W, CHUNK)
    dpad = N + (jnp.arange(pad, dtype=jnp.int32) % (ROWS_PAD - N))
    dstb = jnp.concatenate([dst, dpad]).reshape(NW, CH_PER_W, CHUNK)
    zf = jnp.zeros((NC, ROWS_PAD, D), _f32)
    zc = jnp.zeros((NC, ROWS_PAD), _f32)

    agg1, cnt = _sc_agg_counts(x, srcb, dstb, zf, zc)
    c0 = cnt[0, :N, None]
    c1 = cnt[1, :N, None]
    h = _tc1(x, agg1, agg1, c0, c1,
             Wl1.T, bl1[None, :], Wr1.T, Wlin.T, blin[None, :])

    (agg2,) = _sc_agg(h, srcb, dstb, zf, zc)
    wlin2t = jnp.zeros((D, D), _f32).at[:, :2].set(Wlin2.T)
    blin2p = jnp.full((1, D), -1e30, _f32).at[0, :2].set(blin2)
    y, p = _tc2(h, agg2, agg2, c0, c1,
                Wl2.T, bl2[None, :], Wr2.T, wlin2t, blin2p)
    return (p, y)
